# baseline (speedup 1.0000x reference)
import jax
import jax.numpy as jnp
from jax.experimental import pallas as pl
from jax.experimental.pallas import tpu as pltpu

_K = 10
_MOM = 0.9
_EPS = 1e-8
_B = 128
_BLK = 16


def _dotT(x, w):
    return jax.lax.dot_general(x, w, (((1,), (1,)), ((), ())),
                               preferred_element_type=jnp.float32)


def _topk_mask(sim):
    b = sim.shape[1]
    col = jax.lax.broadcasted_iota(jnp.int32, sim.shape, 1)
    mask = jnp.zeros(sim.shape, jnp.float32)
    work = sim
    for _ in range(_K):
        m = jnp.max(work, axis=1, keepdims=True)
        cand = jnp.where(work == m, col, b)
        amin = jnp.min(cand, axis=1, keepdims=True)
        pick = col == amin
        mask = jnp.where(pick, 1.0, mask)
        work = jnp.where(pick, -jnp.inf, work)
    return mask


def _sim_masked(x):
    nrm = jnp.sqrt(jnp.sum(x * x, axis=1, keepdims=True))
    n = x / jnp.maximum(nrm, 1e-12)
    sim = _dotT(n, n)
    r = jax.lax.broadcasted_iota(jnp.int32, sim.shape, 0)
    c = jax.lax.broadcasted_iota(jnp.int32, sim.shape, 1)
    sim = jnp.where(r == c, -1.0, sim)
    return _topk_mask(sim)


def _normalize_rows(w):
    return (w + _EPS / _B) / (jnp.sum(w, axis=1, keepdims=True) + _EPS)


def _prep_kernel(fer_ref, lm_ref, logits_ref, idx_ref, bank_ref,
                 fwin_ref, fbin_ref, fw1_ref, fb1_ref,
                 lwin_ref, lbin_ref, lw1_ref, lb1_ref,
                 lw2_ref, lb2_ref, lw3_ref, lb3_ref,
                 a_ref, bm_ref, maskf_ref, partial_ref, probs_ref):
    fer = fer_ref[...]
    lm = lm_ref[...]

    maskf_ref[...] = _sim_masked(fer)
    mask_lm = _sim_masked(lm)

    lg = logits_ref[...]
    e = jnp.exp(lg - jnp.max(lg, axis=1, keepdims=True))
    probs = e / jnp.sum(e, axis=1, keepdims=True)
    probs_ref[...] = probs

    bf = jnp.bfloat16
    h = (_dotT(fer.astype(bf), fwin_ref[...]) + fbin_ref[...]).astype(bf)
    cf = h.shape[1] // 2
    w1 = fw1_ref[...]
    a_ref[...] = _dotT(h[:, :cf], w1[:, :cf]).astype(bf)
    bm_ref[...] = (_dotT(h[:, cf:], w1[:, cf:]) + fb1_ref[...]).astype(bf)

    hl = (_dotT(lm.astype(bf), lwin_ref[...]) + lbin_ref[...]).astype(bf)
    cl = hl.shape[1] // 2
    w1l = lw1_ref[...]
    al = _dotT(hl[:, :cl], w1l[:, :cl]).astype(bf)
    bl = (_dotT(hl[:, cl:], w1l[:, cl:]) + lb1_ref[...]).astype(bf)
    h1 = jnp.maximum(al[:, None, :] + bl[None, :, :], jnp.asarray(0.0, bf))
    h1 = h1.reshape(_B * _B, cl)
    h2 = jnp.maximum(_dotT(h1, lw2_ref[...]) + lb2_ref[...], 0.0)
    w3 = lw3_ref[...]
    w3d = w3[0:1, :] - w3[1:2, :]
    b3 = lb3_ref[...]
    b3d = b3[0, 0] - b3[0, 1]
    sraw = jnp.sum(h2 * w3d, axis=1, keepdims=True)
    s = jax.nn.sigmoid(sraw + b3d).reshape(_B, _B)
    lm_w = _normalize_rows(s * mask_lm)

    nbank = bank_ref.shape[0]
    oh = (idx_ref[...] == jax.lax.broadcasted_iota(
        jnp.int32, (_B, nbank), 1)).astype(jnp.float32)
    bank_part = jnp.dot(oh, bank_ref[...],
                        preferred_element_type=jnp.float32) * _MOM
    partial_ref[...] = bank_part + (0.5 * (1.0 - _MOM)) * jnp.dot(
        lm_w, probs, preferred_element_type=jnp.float32)


def _fer_kernel(a_ref, bm_ref, fw2_ref, fb2_ref, fw3_ref, fb3_ref,
                maskf_ref, probs_ref, partial_ref, out_ref):
    a = a_ref[...]
    bm = bm_ref[...]
    c2 = a.shape[1]
    h1 = jnp.maximum(a[:, None, :] + bm[None, :, :],
                     jnp.asarray(0.0, jnp.bfloat16))
    h1 = h1.reshape(_BLK * _B, c2)
    h2 = jnp.maximum(_dotT(h1, fw2_ref[...]) + fb2_ref[...], 0.0)
    w3 = fw3_ref[...]
    w3d = w3[0:1, :] - w3[1:2, :]
    b3 = fb3_ref[...]
    b3d = b3[0, 0] - b3[0, 1]
    sraw = jnp.sum(h2 * w3d, axis=1, keepdims=True)
    s = jax.nn.sigmoid(sraw + b3d).reshape(_BLK, _B)
    fer_w = _normalize_rows(s * maskf_ref[...])
    out_ref[...] = partial_ref[...] + (0.5 * (1.0 - _MOM)) * jnp.dot(
        fer_w, probs_ref[...], preferred_element_type=jnp.float32)


def kernel(fer_features, lm_features, logits, idx, bank,
           fer_Win, fer_bin, fer_W1, fer_b1, fer_W2, fer_b2, fer_W3, fer_b3,
           lm_Win, lm_bin, lm_W1, lm_b1, lm_W2, lm_b2, lm_W3, lm_b3):
    f32 = jnp.float32
    bf = jnp.bfloat16
    idx2 = idx.reshape(_B, 1).astype(jnp.int32)
    row = lambda v: v.reshape(1, -1)

    nc = bank.shape[1]
    c2 = fer_W1.shape[0]

    a, bm, maskf, partial, probs = pl.pallas_call(
        _prep_kernel,
        out_shape=[
            jax.ShapeDtypeStruct((_B, c2), bf),
            jax.ShapeDtypeStruct((_B, c2), bf),
            jax.ShapeDtypeStruct((_B, _B), f32),
            jax.ShapeDtypeStruct((_B, nc), f32),
            jax.ShapeDtypeStruct((_B, nc), f32),
        ],
    )(fer_features, lm_features, logits, idx2, bank,
      fer_Win.astype(bf), row(fer_bin), fer_W1.astype(bf), row(fer_b1),
      lm_Win.astype(bf), row(lm_bin), lm_W1.astype(bf), row(lm_b1),
      lm_W2.astype(bf), row(lm_b2), lm_W3, row(lm_b3))

    nblk = _B // _BLK
    c4 = fer_W2.shape[0]
    full = lambda shape: pl.BlockSpec(shape, lambda i: (0, 0))
    out = pl.pallas_call(
        _fer_kernel,
        grid=(nblk,),
        in_specs=[
            pl.BlockSpec((_BLK, c2), lambda i: (i, 0)),
            full((_B, c2)),
            full((c4, c2)),
            full((1, c4)),
            full((2, c4)),
            full((1, 2)),
            pl.BlockSpec((_BLK, _B), lambda i: (i, 0)),
            full((_B, nc)),
            pl.BlockSpec((_BLK, nc), lambda i: (i, 0)),
        ],
        out_specs=pl.BlockSpec((_BLK, nc), lambda i: (i, 0)),
        out_shape=jax.ShapeDtypeStruct((_B, nc), f32),
    )(a, bm, fer_W2.astype(bf), row(fer_b2), fer_W3, row(fer_b3),
      maskf, probs, partial)
    return out


# speedup vs baseline: 3.2858x; 3.2858x over previous
import jax
import jax.numpy as jnp
from jax.experimental import pallas as pl
from jax.experimental.pallas import tpu as pltpu

_K = 10
_MOM = 0.9
_EPS = 1e-8
_B = 128


def _dotT(x, w):
    return jax.lax.dot_general(x, w, (((1,), (1,)), ((), ())),
                               preferred_element_type=jnp.float32)


def _topk_picks(sim):
    b = sim.shape[1]
    col = jax.lax.broadcasted_iota(jnp.int32, sim.shape, 1)
    picks = []
    work = sim
    for _ in range(_K):
        m = jnp.max(work, axis=1, keepdims=True)
        cand = jnp.where(work == m, col, b)
        amin = jnp.min(cand, axis=1, keepdims=True)
        pick = (col == amin).astype(jnp.float32)
        picks.append(pick)
        work = jnp.where(pick > 0, -jnp.inf, work)
    return picks


def _sim_picks(x):
    nrm = jnp.sqrt(jnp.sum(x * x, axis=1, keepdims=True))
    n = x / jnp.maximum(nrm, 1e-12)
    sim = _dotT(n, n)
    r = jax.lax.broadcasted_iota(jnp.int32, sim.shape, 0)
    c = jax.lax.broadcasted_iota(jnp.int32, sim.shape, 1)
    sim = jnp.where(r == c, -1.0, sim)
    return _topk_picks(sim)


def _selected_scores(x, picks, win, bin_, w1, b1, w2, b2, w3, b3):
    h = _dotT(x, win) + bin_
    c = h.shape[1]
    a = _dotT(h[:, :c // 2], w1[:, :c // 2])
    bm = _dotT(h[:, c // 2:], w1[:, c // 2:]) + b1
    w3d = w3[0:1, :] - w3[1:2, :]
    b3d = b3[0, 0] - b3[0, 1]
    s_full = jnp.zeros((_B, _B), jnp.float32)
    for pick in picks:
        bsel = jnp.dot(pick, bm, preferred_element_type=jnp.float32)
        h1 = jnp.maximum(a + bsel, 0.0)
        h2 = jnp.maximum(_dotT(h1, w2) + b2, 0.0)
        sraw = jnp.sum(h2 * w3d, axis=1, keepdims=True)
        s = jax.nn.sigmoid(sraw + b3d)
        s_full = s_full + s * pick
    return s_full


def _normalize_rows(w):
    return (w + _EPS / _B) / (jnp.sum(w, axis=1, keepdims=True) + _EPS)


def _kernel(fer_ref, lm_ref, logits_ref, idx_ref, bank_ref,
            fwin_ref, fbin_ref, fw1_ref, fb1_ref, fw2_ref, fb2_ref,
            fw3_ref, fb3_ref,
            lwin_ref, lbin_ref, lw1_ref, lb1_ref, lw2_ref, lb2_ref,
            lw3_ref, lb3_ref, out_ref):
    fer_picks = _sim_picks(fer_ref[...])
    lm_picks = _sim_picks(lm_ref[...])

    fer_s = _selected_scores(fer_ref[...], fer_picks,
                             fwin_ref[...], fbin_ref[...], fw1_ref[...],
                             fb1_ref[...], fw2_ref[...], fb2_ref[...],
                             fw3_ref[...], fb3_ref[...])
    lm_s = _selected_scores(lm_ref[...], lm_picks,
                            lwin_ref[...], lbin_ref[...], lw1_ref[...],
                            lb1_ref[...], lw2_ref[...], lb2_ref[...],
                            lw3_ref[...], lb3_ref[...])

    fer_w = _normalize_rows(fer_s)
    lm_w = _normalize_rows(lm_s)

    lg = logits_ref[...]
    e = jnp.exp(lg - jnp.max(lg, axis=1, keepdims=True))
    probs = e / jnp.sum(e, axis=1, keepdims=True)

    nbank = bank_ref.shape[0]
    oh = (idx_ref[...] == jax.lax.broadcasted_iota(
        jnp.int32, (_B, nbank), 1)).astype(jnp.float32)
    bank_part = jnp.dot(oh, bank_ref[...],
                        preferred_element_type=jnp.float32) * _MOM

    out_ref[...] = bank_part + (0.5 * (1.0 - _MOM)) * jnp.dot(
        fer_w + lm_w, probs, preferred_element_type=jnp.float32)


def kernel(fer_features, lm_features, logits, idx, bank,
           fer_Win, fer_bin, fer_W1, fer_b1, fer_W2, fer_b2, fer_W3, fer_b3,
           lm_Win, lm_bin, lm_W1, lm_b1, lm_W2, lm_b2, lm_W3, lm_b3):
    idx2 = idx.reshape(_B, 1).astype(jnp.int32)
    row = lambda v: v.reshape(1, -1)
    nc = bank.shape[1]
    return pl.pallas_call(
        _kernel,
        out_shape=jax.ShapeDtypeStruct((_B, nc), jnp.float32),
    )(fer_features, lm_features, logits, idx2, bank,
      fer_Win, row(fer_bin), fer_W1, row(fer_b1), fer_W2, row(fer_b2),
      fer_W3, row(fer_b3),
      lm_Win, row(lm_bin), lm_W1, row(lm_b1), lm_W2, row(lm_b2),
      lm_W3, row(lm_b3))


# speedup vs baseline: 3.6779x; 1.1193x over previous
import jax
import jax.numpy as jnp
from jax.experimental import pallas as pl
from jax.experimental.pallas import tpu as pltpu

_K = 10
_MOM = 0.9
_EPS = 1e-8
_B = 128


def _dotT(x, w):
    return jax.lax.dot_general(x, w, (((1,), (1,)), ((), ())),
                               preferred_element_type=jnp.float32)


def _topk_picks(sim):
    b = sim.shape[1]
    col = jax.lax.broadcasted_iota(jnp.int32, sim.shape, 1)
    picks = []
    work = sim
    for _ in range(_K):
        m = jnp.max(work, axis=1, keepdims=True)
        cand = jnp.where(work == m, col, b)
        amin = jnp.min(cand, axis=1, keepdims=True)
        pick = (col == amin).astype(jnp.float32)
        picks.append(pick)
        work = jnp.where(pick > 0, -jnp.inf, work)
    return picks


def _sim_picks(x):
    nrm = jnp.sqrt(jnp.sum(x * x, axis=1, keepdims=True))
    n = x / jnp.maximum(nrm, 1e-12)
    sim = _dotT(n, n)
    r = jax.lax.broadcasted_iota(jnp.int32, sim.shape, 0)
    c = jax.lax.broadcasted_iota(jnp.int32, sim.shape, 1)
    sim = jnp.where(r == c, -1.0, sim)
    return _topk_picks(sim)


def _selected_scores(x, picks, win, bin_, w1, b1, w2, b2, w3, b3):
    h = _dotT(x, win) + bin_
    c = h.shape[1]
    a = _dotT(h[:, :c // 2], w1[:, :c // 2])
    bm = _dotT(h[:, c // 2:], w1[:, c // 2:]) + b1
    w3d = w3[0:1, :] - w3[1:2, :]
    b3d = b3[0, 0] - b3[0, 1]
    pmat = jnp.concatenate(picks, axis=0)
    bsel = jnp.dot(pmat, bm, preferred_element_type=jnp.float32)
    at = jnp.concatenate([a] * _K, axis=0)
    h1 = jnp.maximum(at + bsel, 0.0)
    h2 = jnp.maximum(_dotT(h1, w2) + b2, 0.0)
    sraw = jnp.sum(h2 * w3d, axis=1, keepdims=True)
    s = jax.nn.sigmoid(sraw + b3d)
    s_full = jnp.zeros((_B, _B), jnp.float32)
    for k, pick in enumerate(picks):
        s_full = s_full + s[k * _B:(k + 1) * _B] * pick
    return s_full


def _normalize_rows(w):
    return (w + _EPS / _B) / (jnp.sum(w, axis=1, keepdims=True) + _EPS)


def _kernel(fer_ref, lm_ref, logits_ref, idx_ref, bank_ref,
            fwin_ref, fbin_ref, fw1_ref, fb1_ref, fw2_ref, fb2_ref,
            fw3_ref, fb3_ref,
            lwin_ref, lbin_ref, lw1_ref, lb1_ref, lw2_ref, lb2_ref,
            lw3_ref, lb3_ref, out_ref):
    fer_picks = _sim_picks(fer_ref[...])
    lm_picks = _sim_picks(lm_ref[...])

    fer_s = _selected_scores(fer_ref[...], fer_picks,
                             fwin_ref[...], fbin_ref[...], fw1_ref[...],
                             fb1_ref[...], fw2_ref[...], fb2_ref[...],
                             fw3_ref[...], fb3_ref[...])
    lm_s = _selected_scores(lm_ref[...], lm_picks,
                            lwin_ref[...], lbin_ref[...], lw1_ref[...],
                            lb1_ref[...], lw2_ref[...], lb2_ref[...],
                            lw3_ref[...], lb3_ref[...])

    fer_w = _normalize_rows(fer_s)
    lm_w = _normalize_rows(lm_s)

    lg = logits_ref[...]
    e = jnp.exp(lg - jnp.max(lg, axis=1, keepdims=True))
    probs = e / jnp.sum(e, axis=1, keepdims=True)

    nbank = bank_ref.shape[0]
    oh = (idx_ref[...] == jax.lax.broadcasted_iota(
        jnp.int32, (_B, nbank), 1)).astype(jnp.float32)
    bank_part = jnp.dot(oh, bank_ref[...],
                        preferred_element_type=jnp.float32) * _MOM

    out_ref[...] = bank_part + (0.5 * (1.0 - _MOM)) * jnp.dot(
        fer_w + lm_w, probs, preferred_element_type=jnp.float32)


def kernel(fer_features, lm_features, logits, idx, bank,
           fer_Win, fer_bin, fer_W1, fer_b1, fer_W2, fer_b2, fer_W3, fer_b3,
           lm_Win, lm_bin, lm_W1, lm_b1, lm_W2, lm_b2, lm_W3, lm_b3):
    idx2 = idx.reshape(_B, 1).astype(jnp.int32)
    row = lambda v: v.reshape(1, -1)
    nc = bank.shape[1]
    return pl.pallas_call(
        _kernel,
        out_shape=jax.ShapeDtypeStruct((_B, nc), jnp.float32),
    )(fer_features, lm_features, logits, idx2, bank,
      fer_Win, row(fer_bin), fer_W1, row(fer_b1), fer_W2, row(fer_b2),
      fer_W3, row(fer_b3),
      lm_Win, row(lm_bin), lm_W1, row(lm_b1), lm_W2, row(lm_b2),
      lm_W3, row(lm_b3))
